# baseline (speedup 1.0000x reference)
import functools

import jax
import jax.numpy as jnp
from jax import lax
from jax.experimental import pallas as pl
from jax.experimental.pallas import tpu as pltpu
from jax.experimental.pallas import tpu_sc as plsc

B = 16384
D = 32
VOC = 1000000
QROWS = VOC // 4
EPS = 1e-05
NC = 2
NS = 16
NW = NC * NS
BPW = B // NW
CHUNK = 128
NCHUNK = BPW // CHUNK
LANES = 16
GPC = CHUNK // LANES

_mesh = plsc.VectorSubcoreMesh(core_axis_name="c", subcore_axis_name="s")


@functools.partial(
    pl.kernel,
    mesh=_mesh,
    compiler_params=pltpu.CompilerParams(needs_layout_passes=False),
    out_type=jax.ShapeDtypeStruct((B,), jnp.float32),
    scratch_types=[
        pltpu.VMEM((BPW,), jnp.int32),
        pltpu.VMEM((BPW,), jnp.int32),
        pltpu.VMEM((BPW,), jnp.int32),
        pltpu.VMEM((BPW,), jnp.int32),
        pltpu.VMEM((2, CHUNK, 128), jnp.float32),
        pltpu.VMEM((2, CHUNK, 128), jnp.float32),
        pltpu.VMEM((BPW,), jnp.float32),
        pltpu.SemaphoreType.DMA,
        pltpu.SemaphoreType.DMA,
    ],
)
def _gamma_sc(lidx_hbm, ridx_hbm, table_hbm, out_hbm,
              lidx_v, ridx_v, ql_v, qr_v, u_v, v_v, g_v, sem0, sem1):
    wid = lax.axis_index("s") * NC + lax.axis_index("c")
    base = wid * BPW
    pltpu.sync_copy(lidx_hbm.at[pl.ds(base, BPW)], lidx_v)
    pltpu.sync_copy(ridx_hbm.at[pl.ds(base, BPW)], ridx_v)
    for k in range(BPW // LANES):
        s = pl.ds(k * LANES, LANES)
        ql_v[s] = lax.shift_right_logical(lidx_v[s], 2)
        qr_v[s] = lax.shift_right_logical(ridx_v[s], 2)

    sems = (sem0, sem1)

    def fire(j):
        buf = j % 2
        sl = pl.ds(j * CHUNK, CHUNK)
        cl = pltpu.async_copy(table_hbm.at[ql_v.at[sl]], u_v.at[buf], sems[buf])
        cr = pltpu.async_copy(table_hbm.at[qr_v.at[sl]], v_v.at[buf], sems[buf])
        return cl, cr

    lanes = lax.iota(jnp.int32, LANES)
    inflight = fire(0)

    for j in range(NCHUNK):
        if j + 1 < NCHUNK:
            nxt = fire(j + 1)
        inflight[0].wait()
        inflight[1].wait()
        buf = j % 2

        def body(g, carry):
            off = j * CHUNK + g * LANES
            rows = g * LANES + lanes
            cbl = (lidx_v[pl.ds(off, LANES)] & 3) * D
            cbr = (ridx_v[pl.ds(off, LANES)] & 3) * D
            uu = jnp.zeros((LANES,), jnp.float32)
            vv = jnp.zeros((LANES,), jnp.float32)
            uv = jnp.zeros((LANES,), jnp.float32)
            for dcol in range(D):
                gu = plsc.load_gather(u_v.at[buf], [rows, cbl + dcol])
                gv = plsc.load_gather(v_v.at[buf], [rows, cbr + dcol])
                uu = uu + gu * gu
                vv = vv + gv * gv
                uv = uv + gu * gv
            alpha = 1.0 - uu
            alpha = jnp.where(alpha <= 0.0, EPS, alpha)
            beta = 1.0 - vv
            beta = jnp.where(beta <= 0.0, EPS, beta)
            gamma = 1.0 + 2.0 * (uu - 2.0 * uv + vv) / alpha / beta
            gamma = jnp.maximum(gamma, 1.0)
            g_v[pl.ds(off, LANES)] = gamma
            return carry

        lax.fori_loop(0, GPC, body, 0)
        if j + 1 < NCHUNK:
            inflight = nxt

    pltpu.sync_copy(g_v, out_hbm.at[pl.ds(base, BPW)])


def _arcosh_body(g_ref, o_ref):
    g = g_ref[...]
    o_ref[...] = jnp.log(g + jnp.sqrt(g * g - 1.0))


def _arcosh(gamma2d):
    return pl.pallas_call(
        _arcosh_body,
        out_shape=jax.ShapeDtypeStruct(gamma2d.shape, jnp.float32),
    )(gamma2d)


def kernel(left_idx, right_idx, table):
    lidx = left_idx.astype(jnp.int32)
    ridx = right_idx.astype(jnp.int32)
    qtab = table.reshape(QROWS, 128)
    gamma = _gamma_sc(lidx, ridx, qtab)
    dists = _arcosh(gamma.reshape(128, 128))
    return dists.reshape(B)
